# baseline (speedup 1.0000x reference)
import functools

import jax
import jax.numpy as jnp
from jax import lax
from jax.experimental import pallas as pl
from jax.experimental.pallas import tpu as pltpu
from jax.experimental.pallas import tpu_sc as plsc


def _mm_relu_body(x_ref, w_ref, b_ref, o_ref):
    acc = jnp.dot(x_ref[...], w_ref[...], preferred_element_type=jnp.float32)
    o_ref[...] = jnp.maximum(acc + b_ref[...], 0.0)


def _mm_relu(x, w, b, bm=2000):
    n, d = x.shape
    l = w.shape[1]
    return pl.pallas_call(
        _mm_relu_body,
        grid=(n // bm,),
        in_specs=[
            pl.BlockSpec((bm, d), lambda i: (i, 0)),
            pl.BlockSpec((d, l), lambda i: (0, 0)),
            pl.BlockSpec((1, l), lambda i: (0, 0)),
        ],
        out_specs=pl.BlockSpec((bm, l), lambda i: (i, 0)),
        out_shape=jax.ShapeDtypeStruct((n, l), jnp.float32),
    )(x, w, b)


def _latent_body(h_ref, w_ref, b_ref, o_ref):
    acc = jnp.dot(h_ref[0], w_ref[0], preferred_element_type=jnp.float32)
    o_ref[...] = (acc + b_ref[0])[None]


def _latent_mm(hs, ws, bs, bm=2000):
    _, n, k = hs.shape
    l = ws.shape[2]
    return pl.pallas_call(
        _latent_body,
        grid=(3, n // bm),
        in_specs=[
            pl.BlockSpec((1, bm, k), lambda j, i: (j, i, 0)),
            pl.BlockSpec((1, k, l), lambda j, i: (j, 0, 0)),
            pl.BlockSpec((1, 1, l), lambda j, i: (j, 0, 0)),
        ],
        out_specs=pl.BlockSpec((1, bm, l), lambda j, i: (j, i, 0)),
        out_shape=jax.ShapeDtypeStruct((3, n, l), jnp.float32),
    )(hs, ws, bs)


def _bilinear_body(w_ref, p1_ref, p2_ref, o_ref):
    t2 = jnp.dot(w_ref[0], p2_ref[...], preferred_element_type=jnp.float32)
    o_ref[...] = jnp.sum(p1_ref[...] * t2, axis=0, keepdims=True)[None]


def _bilinear(w_bil, p1t, p2t):
    h1, a, _ = w_bil.shape
    b = p1t.shape[1]
    return pl.pallas_call(
        _bilinear_body,
        grid=(h1,),
        in_specs=[
            pl.BlockSpec((1, a, a), lambda h: (h, 0, 0)),
            pl.BlockSpec((a, b), lambda h: (0, 0)),
            pl.BlockSpec((a, b), lambda h: (0, 0)),
        ],
        out_specs=pl.BlockSpec((1, 1, b), lambda h: (h, 0, 0)),
        out_shape=jax.ShapeDtypeStruct((h1, 1, b), jnp.float32),
    )(w_bil, p1t, p2t)


def _elu(x):
    return jnp.where(x > 0, x, jnp.exp(jnp.minimum(x, 0.0)) - 1.0)


def _head_body(f_ref, bb_ref, w1_ref, b1_ref, w2_ref, b2_ref, o_ref):
    f = _elu(f_ref[...] + bb_ref[...])
    h = _elu(jnp.dot(f, w1_ref[...], preferred_element_type=jnp.float32)
             + b1_ref[...])
    o_ref[...] = (jnp.dot(h, w2_ref[...], preferred_element_type=jnp.float32)
                  + b2_ref[...])


def _head(feat, b_bil, w1, b1, w2, b2):
    b, h1 = feat.shape
    return pl.pallas_call(
        _head_body,
        in_specs=[pl.BlockSpec(feat.shape, lambda: (0, 0)),
                  pl.BlockSpec((1, h1), lambda: (0, 0)),
                  pl.BlockSpec(w1.shape, lambda: (0, 0)),
                  pl.BlockSpec((1, w1.shape[1]), lambda: (0, 0)),
                  pl.BlockSpec(w2.shape, lambda: (0, 0)),
                  pl.BlockSpec((1, 1), lambda: (0, 0))],
        out_specs=pl.BlockSpec((b, 1), lambda: (0, 0)),
        out_shape=jax.ShapeDtypeStruct((b, 1), jnp.float32),
    )(feat, b_bil, w1, b1, w2, b2)


def _spmm(src, dst, vals, n, x):
    return jax.ops.segment_sum(x[src] * vals[:, None], dst, num_segments=n)


def kernel(features, adj_indices, adj_values, idx,
           W_u1, b_u1, W_u2, b_u2, W_u3, b_u3,
           W_b1, b_b1, W_b2, b_b2, W_b3, b_b3,
           W_bil, b_bil, W_d1, b_d1, W_d2, b_d2):
    n = features.shape[0]
    src = adj_indices[1]
    dst = adj_indices[0]

    w_cat = jnp.concatenate([W_u1, W_u2, W_u3], axis=1)
    b_cat = jnp.concatenate([b_u1, b_u2, b_u3], axis=1)
    u = _mm_relu(features, w_cat, b_cat)

    p1 = _spmm(src, dst, adj_values, n, u[:, 200:600])
    u3 = _spmm(src, dst, adj_values, n, p1[:, 200:400])
    af1 = jnp.concatenate([u[:, 0:200], p1[:, 0:200], u3], axis=1)

    s1 = _spmm(src, dst, adj_values, n, af1)
    s2 = _spmm(src, dst, adj_values, n, s1)

    hs = jnp.stack([af1, s1, s2])
    ws = jnp.stack([W_b1, W_b2, W_b3])
    bs = jnp.stack([b_b1, b_b2, b_b3])
    lat3 = _latent_mm(hs, ws, bs)
    latent = jnp.transpose(lat3, (1, 0, 2)).reshape(n, -1)

    p1t = latent[idx[0]].T
    p2t = latent[idx[1]].T
    feat_t = _bilinear(W_bil, p1t, p2t)[:, 0, :]

    predictions = _head(feat_t.T, b_bil.reshape(1, -1),
                        W_d1, b_d1.reshape(1, -1),
                        W_d2, b_d2.reshape(1, 1))
    return (predictions, latent)


# speedup vs baseline: 3.0333x; 3.0333x over previous
import functools

import jax
import jax.numpy as jnp
from jax import lax
from jax.experimental import pallas as pl
from jax.experimental.pallas import tpu as pltpu
from jax.experimental.pallas import tpu_sc as plsc

N = 10000
E = 320000
EPAD = 321536
EP = EPAD // 16
NB = EP // 128
NROW = 624
NTAIL = N - 16 * NROW

_MESH = plsc.VectorSubcoreMesh(core_axis_name="c", subcore_axis_name="s")


def _spmm_body(fc, src_hbm, dst_hbm, vals_hbm, x_hbm, zeros_hbm, out_hbm,
               acc, srcb, dstb, valsb, rows, sem):
    c = lax.axis_index("c")
    s = lax.axis_index("s")
    r0 = s * NROW
    pltpu.sync_copy(zeros_hbm.at[pl.ds(r0, NROW)], acc.at[pl.ds(r0, NROW)])

    @pl.when(s == 15)
    def _():
        pltpu.sync_copy(zeros_hbm.at[pl.ds(16 * NROW, NTAIL)],
                        acc.at[pl.ds(16 * NROW, NTAIL)])

    plsc.subcore_barrier()

    ebase = s * EP

    def batch(b, carry):
        off = ebase + b * 128
        pltpu.sync_copy(src_hbm.at[pl.ds(c * EPAD + off, 128)], srcb)
        pltpu.sync_copy(dst_hbm.at[pl.ds(off, 128)], dstb)
        pltpu.sync_copy(vals_hbm.at[pl.ds(off, 128)], valsb)
        pltpu.async_copy(x_hbm.at[srcb], rows, sem).wait()

        def group(g, carry2):
            vv = valsb[pl.ds(g * 16, 16)]
            for l in range(16):
                bv = jnp.broadcast_to(vv[l], (16,))
                e = g * 16 + l
                for j in range(fc // 16):
                    rows[e, pl.ds(j * 16, 16)] = rows[e, pl.ds(j * 16, 16)] * bv
            return carry2

        lax.fori_loop(0, 8, group, 0)
        pltpu.sync_copy(rows, acc.at[dstb], add=True)
        return carry

    lax.fori_loop(0, NB, batch, 0)
    plsc.subcore_barrier()
    pltpu.sync_copy(acc.at[pl.ds(r0, NROW)],
                    out_hbm.at[pl.ds(c * N + r0, NROW)])

    @pl.when(s == 15)
    def _():
        pltpu.sync_copy(acc.at[pl.ds(16 * NROW, NTAIL)],
                        out_hbm.at[pl.ds(c * N + 16 * NROW, NTAIL)])


_spmm_128 = pl.kernel(
    functools.partial(_spmm_body, 128),
    mesh=_MESH,
    out_type=jax.ShapeDtypeStruct((2 * N, 128), jnp.float32),
    scratch_types=[
        pltpu.VMEM_SHARED((N, 128), jnp.float32),
        pltpu.VMEM((128,), jnp.int32),
        pltpu.VMEM((128,), jnp.int32),
        pltpu.VMEM((128,), jnp.float32),
        pltpu.VMEM((128, 128), jnp.float32),
        pltpu.SemaphoreType.DMA,
    ],
)


def _gather_body(latent_hbm, idx_hbm, out_hbm, idx_v, rows_v, sem):
    wid = lax.axis_index("s") * 2 + lax.axis_index("c")
    base = wid * 64
    pltpu.sync_copy(idx_hbm.at[pl.ds(base, 64)], idx_v)
    pltpu.async_copy(latent_hbm.at[idx_v], rows_v, sem).wait()
    pltpu.sync_copy(rows_v, out_hbm.at[pl.ds(base, 64)])


_pair_gather = pl.kernel(
    _gather_body,
    mesh=_MESH,
    out_type=jax.ShapeDtypeStruct((2048, 640), jnp.float32),
    scratch_types=[
        pltpu.VMEM((64,), jnp.int32),
        pltpu.VMEM((64, 640), jnp.float32),
        pltpu.SemaphoreType.DMA,
    ],
)


def _mm_relu_body(x_ref, w_ref, b_ref, o_ref):
    acc = jnp.dot(x_ref[...], w_ref[...], preferred_element_type=jnp.float32)
    o_ref[...] = jnp.maximum(acc + b_ref[...], 0.0)


def _mm_relu(x, w, b, bm=2000):
    n, d = x.shape
    l = w.shape[1]
    return pl.pallas_call(
        _mm_relu_body,
        grid=(n // bm,),
        in_specs=[
            pl.BlockSpec((bm, d), lambda i: (i, 0)),
            pl.BlockSpec((d, l), lambda i: (0, 0)),
            pl.BlockSpec((1, l), lambda i: (0, 0)),
        ],
        out_specs=pl.BlockSpec((bm, l), lambda i: (i, 0)),
        out_shape=jax.ShapeDtypeStruct((n, l), jnp.float32),
    )(x, w, b)


def _mm_body(x_ref, w_ref, o_ref):
    o_ref[...] = jnp.dot(x_ref[...], w_ref[...],
                         preferred_element_type=jnp.float32)


def _mm(x, w, bm=2000):
    n, d = x.shape
    l = w.shape[1]
    return pl.pallas_call(
        _mm_body,
        grid=(n // bm,),
        in_specs=[
            pl.BlockSpec((bm, d), lambda i: (i, 0)),
            pl.BlockSpec((d, l), lambda i: (0, 0)),
        ],
        out_specs=pl.BlockSpec((bm, l), lambda i: (i, 0)),
        out_shape=jax.ShapeDtypeStruct((n, l), jnp.float32),
    )(x, w)


def _latent_body(af_ref, w_ref, b1_ref, p2_ref, p3_ref, b2_ref, b3_ref, o_ref):
    c1 = jnp.dot(af_ref[...], w_ref[...],
                 preferred_element_type=jnp.float32) + b1_ref[...]
    o_ref[...] = jnp.concatenate(
        [c1, p2_ref[...] + b2_ref[...], p3_ref[...] + b3_ref[...]], axis=1)


def _latent_mm(af1, w1, b1, b2pre, b3pre, b2, b3, bm=2000):
    n = af1.shape[0]
    return pl.pallas_call(
        _latent_body,
        grid=(n // bm,),
        in_specs=[
            pl.BlockSpec((bm, 600), lambda i: (i, 0)),
            pl.BlockSpec((600, 200), lambda i: (0, 0)),
            pl.BlockSpec((1, 200), lambda i: (0, 0)),
            pl.BlockSpec((bm, 200), lambda i: (i, 0)),
            pl.BlockSpec((bm, 200), lambda i: (i, 0)),
            pl.BlockSpec((1, 200), lambda i: (0, 0)),
            pl.BlockSpec((1, 200), lambda i: (0, 0)),
        ],
        out_specs=pl.BlockSpec((bm, 600), lambda i: (i, 0)),
        out_shape=jax.ShapeDtypeStruct((n, 600), jnp.float32),
    )(af1, w1, b1, b2pre, b3pre, b2, b3)


def _bilinear_body(w_ref, p1_ref, p2_ref, o_ref):
    t2 = jnp.dot(w_ref[0], p2_ref[...], preferred_element_type=jnp.float32)
    o_ref[...] = jnp.sum(p1_ref[...] * t2, axis=0, keepdims=True)[None]


def _bilinear(w_bil, p1t, p2t):
    h1, a, _ = w_bil.shape
    b = p1t.shape[1]
    return pl.pallas_call(
        _bilinear_body,
        grid=(h1,),
        in_specs=[
            pl.BlockSpec((1, a, a), lambda h: (h, 0, 0)),
            pl.BlockSpec((a, b), lambda h: (0, 0)),
            pl.BlockSpec((a, b), lambda h: (0, 0)),
        ],
        out_specs=pl.BlockSpec((1, 1, b), lambda h: (h, 0, 0)),
        out_shape=jax.ShapeDtypeStruct((h1, 1, b), jnp.float32),
    )(w_bil, p1t, p2t)


def _elu(x):
    return jnp.where(x > 0, x, jnp.exp(jnp.minimum(x, 0.0)) - 1.0)


def _head_body(f_ref, bb_ref, w1_ref, b1_ref, w2_ref, b2_ref, o_ref):
    f = _elu(f_ref[...] + bb_ref[...])
    h = _elu(jnp.dot(f, w1_ref[...], preferred_element_type=jnp.float32)
             + b1_ref[...])
    o_ref[...] = (jnp.dot(h, w2_ref[...], preferred_element_type=jnp.float32)
                  + b2_ref[...])


def _head(feat, b_bil, w1, b1, w2, b2):
    b, h1 = feat.shape
    return pl.pallas_call(
        _head_body,
        in_specs=[pl.BlockSpec(feat.shape, lambda: (0, 0)),
                  pl.BlockSpec((1, h1), lambda: (0, 0)),
                  pl.BlockSpec(w1.shape, lambda: (0, 0)),
                  pl.BlockSpec((1, w1.shape[1]), lambda: (0, 0)),
                  pl.BlockSpec(w2.shape, lambda: (0, 0)),
                  pl.BlockSpec((1, 1), lambda: (0, 0))],
        out_specs=pl.BlockSpec((b, 1), lambda: (0, 0)),
        out_shape=jax.ShapeDtypeStruct((b, 1), jnp.float32),
    )(feat, b_bil, w1, b1, w2, b2)


def kernel(features, adj_indices, adj_values, idx,
           W_u1, b_u1, W_u2, b_u2, W_u3, b_u3,
           W_b1, b_b1, W_b2, b_b2, W_b3, b_b3,
           W_bil, b_bil, W_d1, b_d1, W_d2, b_d2):
    pad_e = EPAD - E
    src = jnp.concatenate([adj_indices[1].astype(jnp.int32),
                           jnp.zeros((pad_e,), jnp.int32)])
    src2 = jnp.concatenate([src, src + N])
    dst = jnp.concatenate([adj_indices[0].astype(jnp.int32),
                           jnp.zeros((pad_e,), jnp.int32)])
    vals = jnp.concatenate([adj_values, jnp.zeros((pad_e,), jnp.float32)])
    z128 = jnp.zeros((N, 128), jnp.float32)

    def spmm256(x):
        x2 = jnp.concatenate([x[:, :128], x[:, 128:256]], axis=0)
        o = _spmm_128(src2, dst, vals, x2, z128)
        return jnp.concatenate([o[:N], o[N:]], axis=1)

    def spmm400(x):
        xp = jnp.pad(x, ((0, 0), (0, 112)))
        return jnp.concatenate(
            [spmm256(xp[:, :256]), spmm256(xp[:, 256:512])], axis=1)[:, :400]

    def spmm200(x):
        return spmm256(jnp.pad(x, ((0, 0), (0, 56))))[:, :200]

    w_cat = jnp.concatenate([W_u1, W_u2, W_u3], axis=1)
    b_cat = jnp.concatenate([b_u1, b_u2, b_u3], axis=1)
    u = _mm_relu(features, w_cat, b_cat)

    r1 = spmm400(u[:, 200:600])
    u2 = r1[:, 0:200]
    u3 = spmm200(r1[:, 200:400])
    af1 = jnp.concatenate([u[:, 0:200], u2, u3], axis=1)

    m = _mm(af1, jnp.concatenate([W_b2, W_b3], axis=1))
    r3 = spmm400(m)
    b2pre = r3[:, 0:200]
    b3pre = spmm200(r3[:, 200:400])

    latent = _latent_mm(af1, W_b1, b_b1, b2pre, b3pre, b_b2, b_b3)

    latent_pad = jnp.pad(latent, ((0, 0), (0, 40)))
    g = _pair_gather(latent_pad,
                     jnp.concatenate([idx[0], idx[1]]).astype(jnp.int32))
    g = g[:, :600]
    p1t = g[:1024].T
    p2t = g[1024:].T
    feat_t = _bilinear(W_bil, p1t, p2t)[:, 0, :]

    predictions = _head(feat_t.T, b_bil.reshape(1, -1),
                        W_d1, b_d1.reshape(1, -1),
                        W_d2, b_d2.reshape(1, 1))
    return (predictions, latent)


# speedup vs baseline: 3.1290x; 1.0316x over previous
import functools

import jax
import jax.numpy as jnp
from jax import lax
from jax.experimental import pallas as pl
from jax.experimental.pallas import tpu as pltpu
from jax.experimental.pallas import tpu_sc as plsc

N = 10000
E = 320000
EPAD = 327680
EP = EPAD // 16
NB = EP // 128
BW = 3 * 128
SEG = 40
NSEG = NB // SEG
PAIRS = SEG // 2
NROW = 624
NTAIL = N - 16 * NROW

_MESH = plsc.VectorSubcoreMesh(core_axis_name="c", subcore_axis_name="s")


def _spmm_body(fc, edata_hbm, x_hbm, zeros_hbm, out_hbm,
               acc, edv, rowsA, rowsB, dstA, dstB, gsA, gsB, ssA, ssB):
    c = lax.axis_index("c")
    s = lax.axis_index("s")
    r0 = s * NROW
    pltpu.sync_copy(zeros_hbm.at[pl.ds(r0, NROW)], acc.at[pl.ds(r0, NROW)])

    @pl.when(s == 15)
    def _():
        pltpu.sync_copy(zeros_hbm.at[pl.ds(16 * NROW, NTAIL)],
                        acc.at[pl.ds(16 * NROW, NTAIL)])

    plsc.subcore_barrier()

    plsc.subcore_barrier()
    tb = (c * 16 + s) * (NB * BW)

    def gather(b, rows, sem):
        pltpu.async_copy(x_hbm.at[edv.at[pl.ds(b * BW, 128)]], rows, sem)

    def wait_gather(b, rows, sem):
        pltpu.make_async_copy(
            x_hbm.at[edv.at[pl.ds(b * BW, 128)]], rows, sem).wait()

    def scatter(b, rows, dstb, sem):
        for g in range(8):
            dstb[pl.ds(g * 16, 16)] = edv[pl.ds(b * BW + 128 + g * 16, 16)]
        pltpu.async_copy(rows, acc.at[dstb], sem, add=True)

    def wait_scatter(rows, dstb, sem):
        pltpu.make_async_copy(rows, acc.at[dstb], sem).wait()

    def scale(b, rows):
        def group(g, carry2):
            vv = jax.lax.bitcast_convert_type(
                edv[pl.ds(b * BW + 256 + g * 16, 16)], jnp.float32)
            for l in range(16):
                bv = jnp.broadcast_to(vv[l], (16,))
                e = g * 16 + l
                for j in range(fc // 16):
                    rows[e, pl.ds(j * 16, 16)] = rows[e, pl.ds(j * 16, 16)] * bv
            return carry2

        lax.fori_loop(0, 8, group, 0)

    def segment(sg, carry):
        pltpu.sync_copy(edata_hbm.at[pl.ds(tb + sg * (SEG * BW), SEG * BW)],
                        edv)
        gather(0, rowsA, gsA)
        gather(1, rowsB, gsB)
        wait_gather(0, rowsA, gsA)
        scale(0, rowsA)
        scatter(0, rowsA, dstA, ssA)
        wait_gather(1, rowsB, gsB)
        scale(1, rowsB)
        scatter(1, rowsB, dstB, ssB)

        def body(k, carry2):
            b0 = 2 * k
            b1 = b0 + 1
            wait_scatter(rowsA, dstA, ssA)
            gather(b0, rowsA, gsA)
            wait_scatter(rowsB, dstB, ssB)
            gather(b1, rowsB, gsB)
            wait_gather(b0, rowsA, gsA)
            scale(b0, rowsA)
            scatter(b0, rowsA, dstA, ssA)
            wait_gather(b1, rowsB, gsB)
            scale(b1, rowsB)
            scatter(b1, rowsB, dstB, ssB)
            return carry2

        lax.fori_loop(1, PAIRS, body, 0)
        wait_scatter(rowsA, dstA, ssA)
        wait_scatter(rowsB, dstB, ssB)
        return carry

    lax.fori_loop(0, NSEG, segment, 0)
    plsc.subcore_barrier()
    pltpu.sync_copy(acc.at[pl.ds(r0, NROW)],
                    out_hbm.at[pl.ds(c * N + r0, NROW)])

    @pl.when(s == 15)
    def _():
        pltpu.sync_copy(acc.at[pl.ds(16 * NROW, NTAIL)],
                        out_hbm.at[pl.ds(c * N + 16 * NROW, NTAIL)])


_spmm_128 = pl.kernel(
    functools.partial(_spmm_body, 128),
    mesh=_MESH,
    out_type=jax.ShapeDtypeStruct((2 * N, 128), jnp.float32),
    scratch_types=[
        pltpu.VMEM_SHARED((N, 128), jnp.float32),
        pltpu.VMEM((SEG * BW,), jnp.int32),
        pltpu.VMEM((128, 128), jnp.float32),
        pltpu.VMEM((128, 128), jnp.float32),
        pltpu.VMEM((128,), jnp.int32),
        pltpu.VMEM((128,), jnp.int32),
        pltpu.SemaphoreType.DMA,
        pltpu.SemaphoreType.DMA,
        pltpu.SemaphoreType.DMA,
        pltpu.SemaphoreType.DMA,
    ],
)


def _gather_body(latent_hbm, idx_hbm, out_hbm, idx_v, rows_v, sem):
    wid = lax.axis_index("s") * 2 + lax.axis_index("c")
    base = wid * 64
    pltpu.sync_copy(idx_hbm.at[pl.ds(base, 64)], idx_v)
    pltpu.async_copy(latent_hbm.at[idx_v], rows_v, sem).wait()
    pltpu.sync_copy(rows_v, out_hbm.at[pl.ds(base, 64)])


_pair_gather = pl.kernel(
    _gather_body,
    mesh=_MESH,
    out_type=jax.ShapeDtypeStruct((2048, 640), jnp.float32),
    scratch_types=[
        pltpu.VMEM((64,), jnp.int32),
        pltpu.VMEM((64, 640), jnp.float32),
        pltpu.SemaphoreType.DMA,
    ],
)


def _mm_relu_body(x_ref, w_ref, b_ref, o_ref):
    acc = jnp.dot(x_ref[...], w_ref[...], preferred_element_type=jnp.float32)
    o_ref[...] = jnp.maximum(acc + b_ref[...], 0.0)


def _mm_relu(x, w, b, bm=2000):
    n, d = x.shape
    l = w.shape[1]
    return pl.pallas_call(
        _mm_relu_body,
        grid=(n // bm,),
        in_specs=[
            pl.BlockSpec((bm, d), lambda i: (i, 0)),
            pl.BlockSpec((d, l), lambda i: (0, 0)),
            pl.BlockSpec((1, l), lambda i: (0, 0)),
        ],
        out_specs=pl.BlockSpec((bm, l), lambda i: (i, 0)),
        out_shape=jax.ShapeDtypeStruct((n, l), jnp.float32),
    )(x, w, b)


def _mm_body(x_ref, w_ref, o_ref):
    o_ref[...] = jnp.dot(x_ref[...], w_ref[...],
                         preferred_element_type=jnp.float32)


def _mm(x, w, bm=2000):
    n, d = x.shape
    l = w.shape[1]
    return pl.pallas_call(
        _mm_body,
        grid=(n // bm,),
        in_specs=[
            pl.BlockSpec((bm, d), lambda i: (i, 0)),
            pl.BlockSpec((d, l), lambda i: (0, 0)),
        ],
        out_specs=pl.BlockSpec((bm, l), lambda i: (i, 0)),
        out_shape=jax.ShapeDtypeStruct((n, l), jnp.float32),
    )(x, w)


def _latent_body(af_ref, w_ref, b1_ref, p2_ref, p3_ref, b2_ref, b3_ref, o_ref):
    c1 = jnp.dot(af_ref[...], w_ref[...],
                 preferred_element_type=jnp.float32) + b1_ref[...]
    o_ref[...] = jnp.concatenate(
        [c1, p2_ref[...] + b2_ref[...], p3_ref[...] + b3_ref[...]], axis=1)


def _latent_mm(af1, w1, b1, b2pre, b3pre, b2, b3, bm=2000):
    n = af1.shape[0]
    return pl.pallas_call(
        _latent_body,
        grid=(n // bm,),
        in_specs=[
            pl.BlockSpec((bm, 600), lambda i: (i, 0)),
            pl.BlockSpec((600, 200), lambda i: (0, 0)),
            pl.BlockSpec((1, 200), lambda i: (0, 0)),
            pl.BlockSpec((bm, 200), lambda i: (i, 0)),
            pl.BlockSpec((bm, 200), lambda i: (i, 0)),
            pl.BlockSpec((1, 200), lambda i: (0, 0)),
            pl.BlockSpec((1, 200), lambda i: (0, 0)),
        ],
        out_specs=pl.BlockSpec((bm, 600), lambda i: (i, 0)),
        out_shape=jax.ShapeDtypeStruct((n, 600), jnp.float32),
    )(af1, w1, b1, b2pre, b3pre, b2, b3)


def _bilinear_body(w_ref, p1_ref, p2_ref, o_ref):
    t2 = jnp.dot(w_ref[0], p2_ref[...], preferred_element_type=jnp.float32)
    o_ref[...] = jnp.sum(p1_ref[...] * t2, axis=0, keepdims=True)[None]


def _bilinear(w_bil, p1t, p2t):
    h1, a, _ = w_bil.shape
    b = p1t.shape[1]
    return pl.pallas_call(
        _bilinear_body,
        grid=(h1,),
        in_specs=[
            pl.BlockSpec((1, a, a), lambda h: (h, 0, 0)),
            pl.BlockSpec((a, b), lambda h: (0, 0)),
            pl.BlockSpec((a, b), lambda h: (0, 0)),
        ],
        out_specs=pl.BlockSpec((1, 1, b), lambda h: (h, 0, 0)),
        out_shape=jax.ShapeDtypeStruct((h1, 1, b), jnp.float32),
    )(w_bil, p1t, p2t)


def _elu(x):
    return jnp.where(x > 0, x, jnp.exp(jnp.minimum(x, 0.0)) - 1.0)


def _head_body(f_ref, bb_ref, w1_ref, b1_ref, w2_ref, b2_ref, o_ref):
    f = _elu(f_ref[...] + bb_ref[...])
    h = _elu(jnp.dot(f, w1_ref[...], preferred_element_type=jnp.float32)
             + b1_ref[...])
    o_ref[...] = (jnp.dot(h, w2_ref[...], preferred_element_type=jnp.float32)
                  + b2_ref[...])


def _head(feat, b_bil, w1, b1, w2, b2):
    b, h1 = feat.shape
    return pl.pallas_call(
        _head_body,
        in_specs=[pl.BlockSpec(feat.shape, lambda: (0, 0)),
                  pl.BlockSpec((1, h1), lambda: (0, 0)),
                  pl.BlockSpec(w1.shape, lambda: (0, 0)),
                  pl.BlockSpec((1, w1.shape[1]), lambda: (0, 0)),
                  pl.BlockSpec(w2.shape, lambda: (0, 0)),
                  pl.BlockSpec((1, 1), lambda: (0, 0))],
        out_specs=pl.BlockSpec((b, 1), lambda: (0, 0)),
        out_shape=jax.ShapeDtypeStruct((b, 1), jnp.float32),
    )(feat, b_bil, w1, b1, w2, b2)


def kernel(features, adj_indices, adj_values, idx,
           W_u1, b_u1, W_u2, b_u2, W_u3, b_u3,
           W_b1, b_b1, W_b2, b_b2, W_b3, b_b3,
           W_bil, b_bil, W_d1, b_d1, W_d2, b_d2):
    pad_e = EPAD - E
    src = jnp.concatenate([adj_indices[1].astype(jnp.int32),
                           jnp.zeros((pad_e,), jnp.int32)])
    dst = jnp.concatenate([adj_indices[0].astype(jnp.int32),
                           jnp.zeros((pad_e,), jnp.int32)])
    vals = jnp.concatenate([adj_values, jnp.zeros((pad_e,), jnp.float32)])
    vbits = jax.lax.bitcast_convert_type(vals, jnp.int32)

    def tiled(a):
        return a.reshape(16, NB, 128)

    blocks = [jnp.stack([tiled(src + c * N), tiled(dst), tiled(vbits)],
                        axis=2) for c in (0, 1)]
    edata = jnp.stack(blocks, axis=0).reshape(-1)
    z128 = jnp.zeros((N, 128), jnp.float32)

    def spmm256(x):
        x2 = jnp.concatenate([x[:, :128], x[:, 128:256]], axis=0)
        o = _spmm_128(edata, x2, z128)
        return jnp.concatenate([o[:N], o[N:]], axis=1)

    def spmm400(x):
        xp = jnp.pad(x, ((0, 0), (0, 112)))
        return jnp.concatenate(
            [spmm256(xp[:, :256]), spmm256(xp[:, 256:512])], axis=1)[:, :400]

    def spmm200(x):
        return spmm256(jnp.pad(x, ((0, 0), (0, 56))))[:, :200]

    w_cat = jnp.concatenate([W_u1, W_u2, W_u3], axis=1)
    b_cat = jnp.concatenate([b_u1, b_u2, b_u3], axis=1)
    u = _mm_relu(features, w_cat, b_cat)

    r1 = spmm400(u[:, 200:600])
    u2 = r1[:, 0:200]
    u3 = spmm200(r1[:, 200:400])
    af1 = jnp.concatenate([u[:, 0:200], u2, u3], axis=1)

    m = _mm(af1, jnp.concatenate([W_b2, W_b3], axis=1))
    r3 = spmm400(m)
    b2pre = r3[:, 0:200]
    b3pre = spmm200(r3[:, 200:400])

    latent = _latent_mm(af1, W_b1, b_b1, b2pre, b3pre, b_b2, b_b3)

    latent_pad = jnp.pad(latent, ((0, 0), (0, 40)))
    g = _pair_gather(latent_pad,
                     jnp.concatenate([idx[0], idx[1]]).astype(jnp.int32))
    g = g[:, :600]
    p1t = g[:1024].T
    p2t = g[1024:].T
    feat_t = _bilinear(W_bil, p1t, p2t)[:, 0, :]

    predictions = _head(feat_t.T, b_bil.reshape(1, -1),
                        W_d1, b_d1.reshape(1, -1),
                        W_d2, b_d2.reshape(1, 1))
    return (predictions, latent)
